# baseline (speedup 1.0000x reference)
import functools
import jax
import jax.numpy as jnp
from jax import lax
from jax.experimental import pallas as pl
from jax.experimental.pallas import tpu as pltpu
from jax.experimental.pallas import tpu_sc as plsc

_HIDDEN = 4096
_EXPERTS = 64
_K = 8
_BT = 1024
_TOKENS = 16384

_NC = 2
_NS = 16
_NW = _NC * _NS
_TPW = _TOKENS // _NW


def _tc_body(x_ref, w_ref, scores_ref):
    x = x_ref[...]
    w = w_ref[...]
    logits = jax.lax.dot_general(
        x, w, (((1,), (1,)), ((), ())),
        preferred_element_type=jnp.float32,
    )
    m = jnp.max(logits, axis=1, keepdims=True)
    e = jnp.exp(logits - m)
    scores_ref[...] = e / jnp.sum(e, axis=1, keepdims=True)


def _tc_scores(x, W):
    tokens = x.shape[0]
    return pl.pallas_call(
        _tc_body,
        grid=(tokens // _BT,),
        in_specs=[
            pl.BlockSpec((_BT, _HIDDEN), lambda i: (i, 0)),
            pl.BlockSpec((_EXPERTS, _HIDDEN), lambda i: (0, 0)),
        ],
        out_specs=pl.BlockSpec((_BT, _EXPERTS), lambda i: (i, 0)),
        out_shape=jax.ShapeDtypeStruct((tokens, _EXPERTS), jnp.float32),
    )(x, W)


def _merge8(ak, av, bk, bv, lo8):
    ck = jnp.where(lo8, ak, jnp.flip(bk, 0))
    cv = jnp.where(lo8, av, jnp.flip(bv, 0))
    return plsc.sort_key_val(ck, cv, descending=True)


@functools.partial(
    pl.kernel,
    mesh=plsc.VectorSubcoreMesh(core_axis_name="c", subcore_axis_name="s"),
    out_type=[
        jax.ShapeDtypeStruct((_TOKENS * _K,), jnp.float32),
        jax.ShapeDtypeStruct((_TOKENS * _K,), jnp.int32),
    ],
    scratch_types=[
        pltpu.VMEM((_TPW * _EXPERTS,), jnp.float32),
        pltpu.VMEM((_TPW * _K + 16,), jnp.float32),
        pltpu.VMEM((_TPW * _K + 16,), jnp.int32),
    ],
    compiler_params=pltpu.CompilerParams(needs_layout_passes=False),
)
def _sc_topk(scores_hbm, w_hbm, i_hbm, s_buf, w_buf, i_buf):
    wid = lax.axis_index("s") * _NC + lax.axis_index("c")
    pltpu.sync_copy(scores_hbm.at[pl.ds(wid * _TPW * _EXPERTS, _TPW * _EXPERTS)],
                    s_buf)
    lane = lax.broadcasted_iota(jnp.int32, (16,), 0)
    lo8 = lane < 8

    def body(t, _):
        base = t * _EXPERTS
        k0, v0 = plsc.sort_key_val(s_buf[pl.ds(base, 16)], lane,
                                   descending=True)
        k1, v1 = plsc.sort_key_val(s_buf[pl.ds(base + 16, 16)], lane + 16,
                                   descending=True)
        k2, v2 = plsc.sort_key_val(s_buf[pl.ds(base + 32, 16)], lane + 32,
                                   descending=True)
        k3, v3 = plsc.sort_key_val(s_buf[pl.ds(base + 48, 16)], lane + 48,
                                   descending=True)
        ka, va = _merge8(k0, v0, k1, v1, lo8)
        kb, vb = _merge8(k2, v2, k3, v3, lo8)
        kf, vf = _merge8(ka, va, kb, vb, lo8)
        plsc.store_compressed(w_buf.at[pl.ds(t * _K, 16)], kf, mask=lo8)
        plsc.store_compressed(i_buf.at[pl.ds(t * _K, 16)], vf, mask=lo8)
        return ()

    lax.fori_loop(0, _TPW, body, (), unroll=4)
    pltpu.sync_copy(w_buf.at[pl.ds(0, _TPW * _K)],
                    w_hbm.at[pl.ds(wid * _TPW * _K, _TPW * _K)])
    pltpu.sync_copy(i_buf.at[pl.ds(0, _TPW * _K)],
                    i_hbm.at[pl.ds(wid * _TPW * _K, _TPW * _K)])


@jax.jit
def kernel(x, W):
    scores = _tc_scores(x, W)
    w_flat, i_flat = _sc_topk(scores.reshape(-1))
    return (scores,
            w_flat.reshape(_TOKENS, _K),
            i_flat.reshape(_TOKENS, _K))


# speedup vs baseline: 1.3649x; 1.3649x over previous
import functools
import jax
import jax.numpy as jnp
from jax.experimental import pallas as pl

_HIDDEN = 4096
_EXPERTS = 64
_K = 8
_BT = 1024


def _router_body(x_ref, w_ref, scores_ref, weights_ref, indices_ref):
    x = x_ref[...]
    w = w_ref[...]
    logits = jax.lax.dot_general(
        x, w, (((1,), (1,)), ((), ())),
        preferred_element_type=jnp.float32,
    )
    m = jnp.max(logits, axis=1, keepdims=True)
    e = jnp.exp(logits - m)
    s = e / jnp.sum(e, axis=1, keepdims=True)
    scores_ref[...] = s

    iota = jax.lax.broadcasted_iota(
        jnp.int32, (_BT, _EXPERTS), 1).astype(jnp.float32)
    work = s
    ws = []
    ids = []
    for _ in range(_K):
        cur = jnp.max(work, axis=1, keepdims=True)
        cand = jnp.where(work == cur, iota, float(_EXPERTS))
        idx = jnp.min(cand, axis=1, keepdims=True)
        ws.append(cur)
        ids.append(idx)
        work = jnp.where(iota == idx, -1.0, work)
    weights_ref[...] = jnp.concatenate(ws, axis=1)
    indices_ref[...] = jnp.concatenate(ids, axis=1).astype(jnp.int32)


@jax.jit
def kernel(x, W):
    tokens = x.shape[0]
    grid = (tokens // _BT,)
    return pl.pallas_call(
        _router_body,
        grid=grid,
        in_specs=[
            pl.BlockSpec((_BT, _HIDDEN), lambda i: (i, 0)),
            pl.BlockSpec((_EXPERTS, _HIDDEN), lambda i: (0, 0)),
        ],
        out_specs=[
            pl.BlockSpec((_BT, _EXPERTS), lambda i: (i, 0)),
            pl.BlockSpec((_BT, _K), lambda i: (i, 0)),
            pl.BlockSpec((_BT, _K), lambda i: (i, 0)),
        ],
        out_shape=[
            jax.ShapeDtypeStruct((tokens, _EXPERTS), jnp.float32),
            jax.ShapeDtypeStruct((tokens, _K), jnp.float32),
            jax.ShapeDtypeStruct((tokens, _K), jnp.int32),
        ],
    )(x, W)


# speedup vs baseline: 1.3682x; 1.0024x over previous
import functools
import jax
import jax.numpy as jnp
from jax.experimental import pallas as pl

_HIDDEN = 4096
_EXPERTS = 64
_K = 8
_BT = 1024
_HH = _HIDDEN // 2


def _router_body(x1_ref, x2_ref, w1_ref, w2_ref,
                 scores_ref, weights_ref, indices_ref):
    logits = jax.lax.dot_general(
        x1_ref[...], w1_ref[...], (((1,), (1,)), ((), ())),
        preferred_element_type=jnp.float32,
    ) + jax.lax.dot_general(
        x2_ref[...], w2_ref[...], (((1,), (1,)), ((), ())),
        preferred_element_type=jnp.float32,
    )
    m = jnp.max(logits, axis=1, keepdims=True)
    e = jnp.exp(logits - m)
    s = e / jnp.sum(e, axis=1, keepdims=True)
    scores_ref[...] = s

    iota = jax.lax.broadcasted_iota(
        jnp.int32, (_BT, _EXPERTS), 1).astype(jnp.float32)
    work = s
    ws = []
    ids = []
    for _ in range(_K):
        cur = jnp.max(work, axis=1, keepdims=True)
        idx = jnp.argmax(work, axis=1)[:, None].astype(jnp.float32)
        ws.append(cur)
        ids.append(idx)
        work = jnp.where(iota == idx, -1.0, work)
    weights_ref[...] = jnp.concatenate(ws, axis=1)
    indices_ref[...] = jnp.concatenate(ids, axis=1).astype(jnp.int32)


@jax.jit
def kernel(x, W):
    tokens = x.shape[0]
    grid = (tokens // _BT,)
    return pl.pallas_call(
        _router_body,
        grid=grid,
        in_specs=[
            pl.BlockSpec((_BT, _HH), lambda i: (i, 0)),
            pl.BlockSpec((_BT, _HH), lambda i: (i, 1)),
            pl.BlockSpec((_EXPERTS, _HH), lambda i: (0, 0)),
            pl.BlockSpec((_EXPERTS, _HH), lambda i: (0, 1)),
        ],
        out_specs=[
            pl.BlockSpec((_BT, _EXPERTS), lambda i: (i, 0)),
            pl.BlockSpec((_BT, _K), lambda i: (i, 0)),
            pl.BlockSpec((_BT, _K), lambda i: (i, 0)),
        ],
        out_shape=[
            jax.ShapeDtypeStruct((tokens, _EXPERTS), jnp.float32),
            jax.ShapeDtypeStruct((tokens, _K), jnp.float32),
            jax.ShapeDtypeStruct((tokens, _K), jnp.int32),
        ],
    )(x, x, W, W)


# speedup vs baseline: 1.4084x; 1.0294x over previous
import functools
import jax
import jax.numpy as jnp
from jax.experimental import pallas as pl
from jax.experimental.pallas import tpu as pltpu

_HIDDEN = 4096
_EXPERTS = 64
_K = 8
_BT = 1024


def _router_body(x_ref, w_ref, scores_ref, weights_ref, indices_ref,
                 lbuf_ref):
    lg = lbuf_ref[...]

    logits = jax.lax.dot_general(
        x_ref[...], w_ref[...], (((1,), (1,)), ((), ())),
        preferred_element_type=jnp.float32,
    )
    lbuf_ref[...] = logits
    m = jnp.max(lg, axis=1, keepdims=True)
    e = jnp.exp(lg - m)
    s = e / jnp.sum(e, axis=1, keepdims=True)
    scores_ref[...] = s

    iota = jax.lax.broadcasted_iota(
        jnp.int32, (_BT, _EXPERTS), 1).astype(jnp.float32)
    work = s
    ws = []
    ids = []
    for _ in range(_K):
        cur = jnp.max(work, axis=1, keepdims=True)
        idx = jnp.argmax(work, axis=1)[:, None].astype(jnp.float32)
        ws.append(cur)
        ids.append(idx)
        work = jnp.where(iota == idx, -1.0, work)
    weights_ref[...] = jnp.concatenate(ws, axis=1)
    indices_ref[...] = jnp.concatenate(ids, axis=1).astype(jnp.int32)


@jax.jit
def kernel(x, W):
    tokens = x.shape[0]
    nblk = tokens // _BT
    last = nblk - 1

    def x_map(i):
        return (jnp.minimum(i, last), 0)

    def out_map(i):
        return (jnp.maximum(i - 1, 0), 0)

    return pl.pallas_call(
        _router_body,
        grid=(nblk + 1,),
        in_specs=[
            pl.BlockSpec((_BT, _HIDDEN), x_map),
            pl.BlockSpec((_EXPERTS, _HIDDEN), lambda i: (0, 0)),
        ],
        out_specs=[
            pl.BlockSpec((_BT, _EXPERTS), out_map),
            pl.BlockSpec((_BT, _K), out_map),
            pl.BlockSpec((_BT, _K), out_map),
        ],
        out_shape=[
            jax.ShapeDtypeStruct((tokens, _EXPERTS), jnp.float32),
            jax.ShapeDtypeStruct((tokens, _K), jnp.float32),
            jax.ShapeDtypeStruct((tokens, _K), jnp.int32),
        ],
        scratch_shapes=[pltpu.VMEM((_BT, _EXPERTS), jnp.float32)],
    )(x, W)


# speedup vs baseline: 1.4085x; 1.0001x over previous
import jax
import jax.numpy as jnp
from jax.experimental import pallas as pl
from jax.experimental.pallas import tpu as pltpu

_HIDDEN = 4096
_EXPERTS = 64
_K = 8
_BT = 1024


def _router_body(x_ref, w_ref, scores_ref, weights_ref, indices_ref,
                 lbuf_ref):
    lg = lbuf_ref[...]

    logits = jax.lax.dot_general(
        x_ref[...], w_ref[...], (((1,), (1,)), ((), ())),
        preferred_element_type=jnp.float32,
    )
    lbuf_ref[...] = logits
    m = jnp.max(lg, axis=1, keepdims=True)
    e = jnp.exp(lg - m)
    s = e / jnp.sum(e, axis=1, keepdims=True)
    scores_ref[...] = s

    iota = jax.lax.broadcasted_iota(
        jnp.int32, (_BT, _EXPERTS), 1).astype(jnp.float32)
    work = s
    ws = []
    ids = []
    for _ in range(_K):
        cur = jnp.max(work, axis=1, keepdims=True)
        idx = jnp.argmax(work, axis=1)[:, None].astype(jnp.float32)
        ws.append(cur)
        ids.append(idx)
        work = jnp.where(iota == idx, -1.0, work)
    weights_ref[...] = jnp.concatenate(ws, axis=1)
    indices_ref[...] = jnp.concatenate(ids, axis=1).astype(jnp.int32)


@jax.jit
def kernel(x, W):
    tokens = x.shape[0]
    nblk = tokens // _BT
    last = nblk - 1

    def x_map(i):
        return (jnp.minimum(i, last), 0)

    def out_map(i):
        return (jnp.maximum(i - 1, 0), 0)

    return pl.pallas_call(
        _router_body,
        grid=(nblk + 1,),
        in_specs=[
            pl.BlockSpec((_BT, _HIDDEN), x_map),
            pl.BlockSpec((_EXPERTS, _HIDDEN), lambda i: (0, 0)),
        ],
        out_specs=[
            pl.BlockSpec((_BT, _EXPERTS), out_map),
            pl.BlockSpec((_BT, _K), out_map),
            pl.BlockSpec((_BT, _K), out_map),
        ],
        out_shape=[
            jax.ShapeDtypeStruct((tokens, _EXPERTS), jnp.float32),
            jax.ShapeDtypeStruct((tokens, _K), jnp.float32),
            jax.ShapeDtypeStruct((tokens, _K), jnp.int32),
        ],
        scratch_shapes=[pltpu.VMEM((_BT, _EXPERTS), jnp.float32)],
    )(x, W)
